# baseline (speedup 1.0000x reference)
import functools

import jax
import jax.numpy as jnp
from jax import lax
from jax.experimental import pallas as pl
from jax.experimental.pallas import tpu as pltpu
from jax.experimental.pallas import tpu_sc as plsc

BATCH = 16384
DIM = 64
NUM_CORES = 2
NUM_SUBCORES = 16
NUM_WORKERS = NUM_CORES * NUM_SUBCORES
B_PER_W = BATCH // NUM_WORKERS
LANES = 128
GRAN = 16
TILE_COLS = 1000000 // LANES
TAIL_BASE = TILE_COLS * LANES
TAIL_ROWS = 1000000 - TAIL_BASE
NBUF = 7

_mesh = plsc.VectorSubcoreMesh(core_axis_name="c", subcore_axis_name="s")


@functools.partial(
    pl.kernel,
    mesh=_mesh,
    out_type=jax.ShapeDtypeStruct((DIM, BATCH), jnp.float32),
    scratch_types=[
        pltpu.VMEM((B_PER_W + GRAN, ), jnp.int32),
        pltpu.VMEM((NBUF, DIM, LANES), jnp.float32),
        pltpu.VMEM((DIM, LANES), jnp.float32),
        pltpu.VMEM((DIM, B_PER_W), jnp.float32),
        [pltpu.SemaphoreType.DMA] * (2 * NBUF),
    ],
    compiler_params=pltpu.CompilerParams(
        use_tc_tiling_on_sc=True, needs_layout_passes=False
    ),
)
def _emb_lookup(idx_hbm, table_t_hbm, tail_hbm, out_hbm, idx_vm, bufs,
                tail_v, staging, sems):
    wid = lax.axis_index("s") * NUM_CORES + lax.axis_index("c")
    base = wid * B_PER_W
    for rr in range(B_PER_W // LANES):
        pltpu.sync_copy(idx_hbm.at[wid, rr], idx_vm.at[pl.ds(rr * LANES, LANES)])
    pltpu.sync_copy(tail_hbm, tail_v)

    def r_of(j):
        return idx_vm[pl.ds(j, GRAN)][0]

    def issue(j, slot):
        r = r_of(j)
        tc = jnp.minimum(r // LANES, TILE_COLS - 1)
        off = pl.multiple_of(tc * LANES, LANES)
        for h in range(2):
            pltpu.async_copy(
                table_t_hbm.at[pl.ds(h * DIM // 2, DIM // 2), pl.ds(off, LANES)],
                bufs.at[slot, pl.ds(h * DIM // 2, DIM // 2)],
                sems[2 * slot + h],
            )

    def drain(slot):
        for h in range(2):
            pltpu.make_async_copy(
                table_t_hbm.at[pl.ds(0, DIM // 2), pl.ds(0, LANES)],
                bufs.at[slot, pl.ds(h * DIM // 2, DIM // 2)],
                sems[2 * slot + h],
            ).wait()

    def extract(j, slot):
        r = r_of(j)
        lane = jnp.broadcast_to(r % LANES, (GRAN,))
        jv = jnp.broadcast_to(j, (GRAN,))
        is_tail = jnp.broadcast_to(r >= TAIL_BASE, (GRAN,))
        for k in range(DIM // GRAN):
            dv = k * GRAN + lax.iota(jnp.int32, GRAN)
            x_main = plsc.load_gather(bufs.at[slot], [dv, lane])
            x_tail = plsc.load_gather(tail_v, [dv, lane])
            x = jnp.where(is_tail, x_tail, x_main)
            plsc.store_scatter(staging, [dv, jv], x)

    for b in range(NBUF):
        issue(b, b)

    def body(g, _):
        for b in range(NBUF):
            j = g * NBUF + b
            drain(b)
            extract(j, b)

            @pl.when(j + NBUF < B_PER_W)
            def _():
                issue(j + NBUF, b)

        return ()

    lax.fori_loop(0, B_PER_W // NBUF, body, ())
    for t in range(B_PER_W % NBUF):
        drain(t)
        extract((B_PER_W // NBUF) * NBUF + t, t)
    pltpu.sync_copy(staging, out_hbm.at[:, pl.ds(pl.multiple_of(base, LANES), B_PER_W)])


def kernel(input, table):
    idx = input.astype(jnp.int32).reshape(NUM_WORKERS, B_PER_W)
    idx3 = jnp.pad(idx, ((0, 0), (0, 512))).reshape(NUM_WORKERS, 8, LANES)
    tail = jnp.pad(table[TAIL_BASE:].T, ((0, 0), (0, LANES - TAIL_ROWS)))
    return _emb_lookup(idx3, table.T, tail).T


# speedup vs baseline: 1.1521x; 1.1521x over previous
import functools

import jax
import jax.numpy as jnp
from jax import lax
from jax.experimental import pallas as pl
from jax.experimental.pallas import tpu as pltpu
from jax.experimental.pallas import tpu_sc as plsc

BATCH = 16384
DIM = 64
NUM_CORES = 2
NUM_SUBCORES = 16
NUM_WORKERS = NUM_CORES * NUM_SUBCORES
B_PER_W = BATCH // NUM_WORKERS
LANES = 128
GRAN = 16
TILE_COLS = 1000000 // LANES
TAIL_BASE = TILE_COLS * LANES
TAIL_ROWS = 1000000 - TAIL_BASE
NBUF = 9
HALF = B_PER_W // 2

_mesh = plsc.VectorSubcoreMesh(core_axis_name="c", subcore_axis_name="s")


@functools.partial(
    pl.kernel,
    mesh=_mesh,
    out_type=jax.ShapeDtypeStruct((DIM, BATCH), jnp.float32),
    scratch_types=[
        pltpu.VMEM((B_PER_W + GRAN, ), jnp.int32),
        pltpu.VMEM((NBUF, DIM, LANES), jnp.float32),
        pltpu.VMEM((DIM, TAIL_ROWS), jnp.float32),
        pltpu.VMEM((DIM, HALF), jnp.float32),
        [pltpu.SemaphoreType.DMA] * NBUF,
    ],
    compiler_params=pltpu.CompilerParams(
        use_tc_tiling_on_sc=True, needs_layout_passes=False
    ),
)
def _emb_lookup(idx_hbm, table_t_hbm, tail_hbm, out_hbm, idx_vm, bufs,
                tail_v, staging, sems):
    wid = lax.axis_index("s") * NUM_CORES + lax.axis_index("c")
    base = wid * B_PER_W
    for rr in range(B_PER_W // LANES):
        pltpu.sync_copy(idx_hbm.at[wid, rr], idx_vm.at[pl.ds(rr * LANES, LANES)])
    pltpu.sync_copy(tail_hbm, tail_v)

    def r_of(j):
        return idx_vm[pl.ds(j, GRAN)][0]

    def issue(j, slot):
        r = r_of(j)
        tc = jnp.minimum(r // LANES, TILE_COLS - 1)
        off = pl.multiple_of(tc * LANES, LANES)
        pltpu.async_copy(
            table_t_hbm.at[:, pl.ds(off, LANES)], bufs.at[slot], sems[slot]
        )

    def drain(slot):
        pltpu.make_async_copy(
            table_t_hbm.at[:, pl.ds(0, LANES)], bufs.at[slot], sems[slot]
        ).wait()

    def extract(j, slot):
        r = r_of(j)
        lane = jnp.broadcast_to(r % LANES, (GRAN,))
        lane_t = jnp.broadcast_to(jnp.minimum(jnp.maximum(r - TAIL_BASE, 0), TAIL_ROWS - 1), (GRAN,))
        jv = jnp.broadcast_to(j % HALF, (GRAN,))
        is_tail = jnp.broadcast_to(r >= TAIL_BASE, (GRAN,))
        for k in range(DIM // GRAN):
            dv = k * GRAN + lax.iota(jnp.int32, GRAN)
            x_main = plsc.load_gather(bufs.at[slot], [dv, lane])
            x_tail = plsc.load_gather(tail_v, [dv, lane_t])
            x = jnp.where(is_tail, x_tail, x_main)
            plsc.store_scatter(staging, [dv, jv], x)

    for b in range(NBUF):
        issue(b, b)

    def body(g, _):
        for b in range(NBUF):
            j = g * NBUF + b
            drain(b)
            extract(j, b)

            @pl.when(j == HALF - 1)
            def _():
                pltpu.sync_copy(
                    staging,
                    out_hbm.at[:, pl.ds(pl.multiple_of(base, LANES), HALF)],
                )

            @pl.when(j + NBUF < B_PER_W)
            def _():
                issue(j + NBUF, b)

        return ()

    lax.fori_loop(0, B_PER_W // NBUF, body, ())
    for t in range(B_PER_W % NBUF):
        jj = (B_PER_W // NBUF) * NBUF + t
        slot = jj % NBUF
        drain(slot)
        extract(jj, slot)
    pltpu.sync_copy(
        staging, out_hbm.at[:, pl.ds(pl.multiple_of(base + HALF, LANES), HALF)]
    )


def kernel(input, table):
    idx = input.astype(jnp.int32).reshape(NUM_WORKERS, B_PER_W)
    idx3 = jnp.pad(idx, ((0, 0), (0, 512))).reshape(NUM_WORKERS, 8, LANES)
    tail = table[TAIL_BASE:].T
    return _emb_lookup(idx3, table.T, tail).T
